# baseline (speedup 1.0000x reference)
import jax
import jax.numpy as jnp
from jax.experimental import pallas as pl
from jax.experimental.pallas import tpu as pltpu

_CB = 256
_NCB = 16
_TOKENS = 2048


def _encode_kernel(x_ref, w_ref, b_ref, out_ref):
    ones = jnp.ones((x_ref.shape[0], 1), jnp.float32)
    x = jnp.concatenate([x_ref[...], ones], axis=1)
    wt = jnp.concatenate([w_ref[...], b_ref[...]], axis=1)
    t = x.shape[0]
    half = _CB // 2
    iota_lo = jax.lax.broadcasted_iota(jnp.int32, (t, half), 1).astype(jnp.float32)
    iota_hi = iota_lo + float(half)
    for j in range(_NCB):
        s = jax.lax.dot_general(
            x,
            wt[j * _CB : (j + 1) * _CB, :],
            (((1,), (1,)), ((), ())),
            preferred_element_type=jnp.float32,
        )
        t0 = s[:, :half]
        t1 = s[:, half:]
        gt = t1 > t0
        bv = jnp.where(gt, t1, t0)
        bi = jnp.where(gt, iota_hi, iota_lo)
        m = jnp.max(bv, axis=1, keepdims=True)
        idx = jnp.min(jnp.where(bv == m, bi, float(_CB)), axis=1, keepdims=True)
        out_ref[:, j : j + 1] = idx.astype(jnp.uint8)


def kernel(x, W, b):
    batch, hw, dim = x.shape
    n = W.shape[0]
    tokens = batch * hw
    xf = x.reshape(tokens, dim)
    b2 = b.reshape(n, 1)
    out = pl.pallas_call(
        _encode_kernel,
        grid=(tokens // _TOKENS,),
        in_specs=[
            pl.BlockSpec((_TOKENS, dim), lambda i: (i, 0)),
            pl.BlockSpec((n, dim), lambda i: (0, 0)),
            pl.BlockSpec((n, 1), lambda i: (0, 0)),
        ],
        out_specs=pl.BlockSpec((_TOKENS, _NCB), lambda i: (i, 0)),
        compiler_params=pltpu.CompilerParams(dimension_semantics=("parallel",)),
        out_shape=jax.ShapeDtypeStruct((tokens, _NCB), jnp.uint8),
    )(xf, W, b2)
    return out.reshape(batch, hw, _NCB)


# speedup vs baseline: 1.1043x; 1.1043x over previous
import jax
import jax.numpy as jnp
from jax.experimental import pallas as pl
from jax.experimental.pallas import tpu as pltpu

_CB = 256
_NCB = 16
_TOKENS = 2048


def _encode_kernel(x_ref, wt_ref, out_ref):
    ones = jnp.ones((x_ref.shape[0], 1), jnp.float32)
    x = jnp.concatenate([x_ref[...], ones], axis=1)
    wt = wt_ref[...]
    t = x.shape[0]
    half = _CB // 2
    iota_lo = jax.lax.broadcasted_iota(jnp.int32, (t, half), 1).astype(jnp.float32)
    iota_hi = iota_lo + float(half)
    for j in range(_NCB):
        s = jax.lax.dot_general(
            x,
            wt[:, j * _CB : (j + 1) * _CB],
            (((1,), (0,)), ((), ())),
            preferred_element_type=jnp.float32,
        )
        t0 = s[:, :half]
        t1 = s[:, half:]
        gt = t1 > t0
        bv = jnp.where(gt, t1, t0)
        bi = jnp.where(gt, iota_hi, iota_lo)
        m = jnp.max(bv, axis=1, keepdims=True)
        idx = jnp.min(jnp.where(bv == m, bi, float(_CB)), axis=1, keepdims=True)
        out_ref[:, j : j + 1] = idx.astype(jnp.uint8)


def kernel(x, W, b):
    batch, hw, dim = x.shape
    n = W.shape[0]
    tokens = batch * hw
    xf = x.reshape(tokens, dim)
    wa = jnp.concatenate([W.T, b.reshape(1, n)], axis=0)
    out = pl.pallas_call(
        _encode_kernel,
        grid=(tokens // _TOKENS,),
        in_specs=[
            pl.BlockSpec((_TOKENS, dim), lambda i: (i, 0)),
            pl.BlockSpec((dim + 1, n), lambda i: (0, 0)),
        ],
        out_specs=pl.BlockSpec((_TOKENS, _NCB), lambda i: (i, 0)),
        compiler_params=pltpu.CompilerParams(dimension_semantics=("arbitrary",)),
        out_shape=jax.ShapeDtypeStruct((tokens, _NCB), jnp.uint8),
    )(xf, wa)
    return out.reshape(batch, hw, _NCB)
